# baseline (speedup 1.0000x reference)
import functools

import jax
import jax.numpy as jnp
from jax import lax
from jax.experimental import pallas as pl
from jax.experimental.pallas import tpu as pltpu
from jax.experimental.pallas import tpu_sc as plsc

VOCAB = 100000
EMBED = 64
B, L = 4096, 200

NC, NS = 2, 16
NW = NC * NS
BC = B // NW
LH, LL = L // 8, 8
EH, EL = EMBED // 8, 8
BH = B // 128

_mesh = plsc.VectorSubcoreMesh(
    core_axis_name="c", subcore_axis_name="s", num_cores=NC, num_subcores=NS
)


@functools.partial(
    pl.kernel,
    out_type=jax.ShapeDtypeStruct((L, EH, BH, EL, 128), jnp.float32),
    mesh=_mesh,
    compiler_params=pltpu.CompilerParams(
        use_tc_tiling_on_sc=False, needs_layout_passes=False
    ),
    scratch_types=[
        pltpu.VMEM((LH, LL, BC), jnp.int32),
        pltpu.VMEM((L, EMBED), jnp.float32),
        pltpu.VMEM((BC, EMBED), jnp.float32),
        pltpu.VMEM((BC, EMBED), jnp.float32),
        pltpu.VMEM((BC, EMBED), jnp.float32),
        pltpu.VMEM((BC, EMBED), jnp.float32),
        pltpu.VMEM((EH, EL, 129), jnp.float32),
        pltpu.VMEM((EH, EL, 129), jnp.float32),
        pltpu.VMEM((EH, EL, 129), jnp.float32),
        pltpu.VMEM((EH, EL, 129), jnp.float32),
        pltpu.SemaphoreType.DMA,
        pltpu.SemaphoreType.DMA,
    ],
)
def _sc_embed(x_hbm, tok_hbm, pos_hbm, out_hbm, xcol_v, pos_v,
              rows0, rows1, rows2, rows3, stg0, stg1, stg2, stg3,
              sem_g, sem_o):
    wid = lax.axis_index("s") * NC + lax.axis_index("c")
    pltpu.sync_copy(x_hbm.at[:, wid], xcol_v)
    pltpu.sync_copy(pos_hbm.at[pl.ds(0, L)], pos_v)
    rows = (rows0, rows1, rows2, rows3)
    stgs = (stg0, stg1, stg2, stg3)
    ND = 4

    iv = lax.iota(jnp.int32, 16)
    ivh = iv // EL
    elv = iv % EL

    def start_gather(l, buf):
        pltpu.async_copy(tok_hbm.at[xcol_v.at[l // 8, l % 8]], buf, sem_g)

    def wait_gather(buf):
        pltpu.make_async_copy(tok_hbm.at[pl.ds(0, BC)], buf, sem_g).wait()

    def wait_out(stg):
        pltpu.make_async_copy(
            stg.at[:, :, pl.ds(0, 128)], out_hbm.at[0, :, wid], sem_o
        ).wait()

    start_gather(0, rows0)
    start_gather(1, rows1)
    start_gather(2, rows2)

    @pl.loop(0, L, step=4)
    def _block(l0):
        for par in range(ND):
            l = l0 + par
            cur, stg = rows[par], stgs[par]

            @pl.when(l >= ND)
            def _free_stg():
                wait_out(stg)

            @pl.when(l + ND - 1 < L)
            def _prefetch():
                start_gather(l + ND - 1, rows[(par + ND - 1) % ND])

            wait_gather(cur)

            pos = [pos_v[l, pl.ds(16 * k, 16)] for k in range(EMBED // 16)]
            ehv = [2 * k + ivh for k in range(EMBED // 16)]

            @pl.loop(0, BC, unroll=8)
            def _row(r):
                rv = jnp.broadcast_to(r, (16,)).astype(jnp.int32)
                for k in range(EMBED // 16):
                    val = cur[r, pl.ds(16 * k, 16)] + pos[k]
                    plsc.store_scatter(stg, [ehv[k], elv, rv], val)

            pltpu.async_copy(
                stg.at[:, :, pl.ds(0, 128)], out_hbm.at[l, :, wid], sem_o
            )

    for s in stgs:
        wait_out(s)


def kernel(x, tok_table, pos_table):
    x4 = x.T.astype(jnp.int32).reshape(LH, LL, BH, 128).transpose(0, 2, 1, 3)
    out5 = _sc_embed(x4, tok_table, pos_table)
    return out5.transpose(2, 4, 0, 1, 3).reshape(B, L, EMBED)


# speedup vs baseline: 1.0191x; 1.0191x over previous
import functools

import jax
import jax.numpy as jnp
from jax import lax
from jax.experimental import pallas as pl
from jax.experimental.pallas import tpu as pltpu
from jax.experimental.pallas import tpu_sc as plsc

VOCAB = 100000
EMBED = 64
B, L = 4096, 200

NC, NS = 2, 16
NW = NC * NS
BC = B // NW
LH, LL = L // 8, 8
EH, EL = EMBED // 8, 8
BH = B // 128

_mesh = plsc.VectorSubcoreMesh(
    core_axis_name="c", subcore_axis_name="s", num_cores=NC, num_subcores=NS
)


@functools.partial(
    pl.kernel,
    out_type=jax.ShapeDtypeStruct((L, EH, BH, EL, 128), jnp.float32),
    mesh=_mesh,
    compiler_params=pltpu.CompilerParams(
        use_tc_tiling_on_sc=False, needs_layout_passes=False
    ),
    scratch_types=[
        pltpu.VMEM((LH, LL, BC), jnp.int32),
        pltpu.VMEM((L, EMBED), jnp.float32),
        pltpu.VMEM((BC, EMBED), jnp.float32),
        pltpu.VMEM((BC, EMBED), jnp.float32),
        pltpu.VMEM((EH, EL, 145), jnp.float32),
        pltpu.VMEM((EH, EL, 145), jnp.float32),
        pltpu.SemaphoreType.DMA,
        pltpu.SemaphoreType.DMA,
    ],
)
def _sc_embed(x_hbm, tok_hbm, pos_hbm, out_hbm, xcol_v, pos_v, rows0, rows1,
              stg0, stg1, sem_g, sem_o):
    wid = lax.axis_index("s") * NC + lax.axis_index("c")
    pltpu.sync_copy(x_hbm.at[:, wid], xcol_v)
    pltpu.sync_copy(pos_hbm.at[pl.ds(0, L)], pos_v)
    rows = (rows0, rows1)
    stgs = (stg0, stg1)

    iv = lax.iota(jnp.int32, 16)
    ivh = iv // EL
    elv = iv % EL

    def start_gather(l, buf):
        pltpu.async_copy(tok_hbm.at[xcol_v.at[l // 8, l % 8]], buf, sem_g)

    def wait_gather(buf):
        pltpu.make_async_copy(tok_hbm.at[pl.ds(0, BC)], buf, sem_g).wait()

    def wait_out(stg):
        pltpu.make_async_copy(
            stg.at[:, :, pl.ds(0, 128)], out_hbm.at[0, :, wid], sem_o
        ).wait()

    start_gather(0, rows0)

    @pl.loop(0, L, step=2)
    def _block(l0):
        for par in range(2):
            l = l0 + par
            cur, stg = rows[par], stgs[par]

            @pl.when(l >= 2)
            def _free_stg():
                wait_out(stg)

            @pl.when(l + 1 < L)
            def _prefetch():
                start_gather(l + 1, rows[1 - par])

            wait_gather(cur)

            pos = [pos_v[l, pl.ds(16 * k, 16)] for k in range(EMBED // 16)]
            ehv = [2 * k + ivh for k in range(EMBED // 16)]

            @pl.loop(0, BC, unroll=4)
            def _row(r):
                rv = jnp.broadcast_to(r, (16,)).astype(jnp.int32)
                for k in range(EMBED // 16):
                    val = cur[r, pl.ds(16 * k, 16)] + pos[k]
                    plsc.store_scatter(stg, [ehv[k], elv, rv], val)

            pltpu.async_copy(
                stg.at[:, :, pl.ds(0, 128)], out_hbm.at[l, :, wid], sem_o
            )

    wait_out(stg0)
    wait_out(stg1)


def kernel(x, tok_table, pos_table):
    x4 = x.T.astype(jnp.int32).reshape(LH, LL, BH, 128).transpose(0, 2, 1, 3)
    out5 = _sc_embed(x4, tok_table, pos_table)
    return out5.transpose(2, 4, 0, 1, 3).reshape(B, L, EMBED)


# speedup vs baseline: 2.0173x; 1.9795x over previous
import functools

import jax
import jax.numpy as jnp
from jax import lax
from jax.experimental import pallas as pl
from jax.experimental.pallas import tpu as pltpu
from jax.experimental.pallas import tpu_sc as plsc

VOCAB = 100000
EMBED = 64
B, L = 4096, 200

NC, NS = 2, 16
NW = NC * NS
BC = B // NW
LH, LL = L // 8, 8
EH, EL = EMBED // 8, 8
BH = B // 128

_mesh = plsc.VectorSubcoreMesh(
    core_axis_name="c", subcore_axis_name="s", num_cores=NC, num_subcores=NS
)


@functools.partial(
    pl.kernel,
    out_type=jax.ShapeDtypeStruct((L, EH, BH, EL, 128), jnp.float32),
    mesh=_mesh,
    compiler_params=pltpu.CompilerParams(
        use_tc_tiling_on_sc=False, needs_layout_passes=False
    ),
    scratch_types=[
        pltpu.VMEM((LH, LL, BC), jnp.int32),
        pltpu.VMEM((L, EMBED), jnp.float32),
        pltpu.VMEM((BC, EMBED), jnp.float32),
        pltpu.VMEM((BC, EMBED), jnp.float32),
        pltpu.VMEM((EH, EL, 145), jnp.float32),
        pltpu.VMEM((EH, EL, 145), jnp.float32),
        pltpu.SemaphoreType.DMA,
        pltpu.SemaphoreType.DMA,
    ],
)
def _sc_embed(x_hbm, tok_hbm, pos_hbm, out_hbm, xcol_v, pos_v, rows0, rows1,
              stg0, stg1, sem_g, sem_o):
    wid = lax.axis_index("s") * NC + lax.axis_index("c")
    pltpu.sync_copy(x_hbm.at[:, wid], xcol_v)
    pltpu.sync_copy(pos_hbm.at[pl.ds(0, L)], pos_v)
    rows = (rows0, rows1)
    stgs = (stg0, stg1)

    iv = lax.iota(jnp.int32, 16)
    ivh = iv // EL
    elv = iv % EL

    def start_gather(l, buf):
        pltpu.async_copy(tok_hbm.at[xcol_v.at[l // 8, l % 8]], buf, sem_g)

    def wait_gather(buf):
        pltpu.make_async_copy(tok_hbm.at[pl.ds(0, BC)], buf, sem_g).wait()

    def wait_out(stg):
        pltpu.make_async_copy(
            stg.at[:, :, pl.ds(0, 128)], out_hbm.at[0, :, wid], sem_o
        ).wait()

    start_gather(0, rows0)

    @pl.loop(0, L, step=2)
    def _block(l0):
        for par in range(2):
            l = l0 + par
            cur, stg = rows[par], stgs[par]

            @pl.when(l >= 2)
            def _free_stg():
                wait_out(stg)

            @pl.when(l + 1 < L)
            def _prefetch():
                start_gather(l + 1, rows[1 - par])

            wait_gather(cur)

            pos = [pos_v[l, pl.ds(16 * k, 16)] for k in range(EMBED // 16)]
            ehv = [2 * k + ivh for k in range(EMBED // 16)]

            @plsc.parallel_loop(0, BC, unroll=8)
            def _row(r):
                rv = jnp.broadcast_to(r, (16,)).astype(jnp.int32)
                for k in range(EMBED // 16):
                    val = cur[r, pl.ds(16 * k, 16)] + pos[k]
                    plsc.store_scatter(stg, [ehv[k], elv, rv], val)

            pltpu.async_copy(
                stg.at[:, :, pl.ds(0, 128)], out_hbm.at[l, :, wid], sem_o
            )

    wait_out(stg0)
    wait_out(stg1)


def kernel(x, tok_table, pos_table):
    x4 = x.T.astype(jnp.int32).reshape(LH, LL, BH, 128).transpose(0, 2, 1, 3)
    out5 = _sc_embed(x4, tok_table, pos_table)
    return out5.transpose(2, 4, 0, 1, 3).reshape(B, L, EMBED)


# speedup vs baseline: 2.4798x; 1.2292x over previous
import functools

import jax
import jax.numpy as jnp
from jax import lax
from jax.experimental import pallas as pl
from jax.experimental.pallas import tpu as pltpu
from jax.experimental.pallas import tpu_sc as plsc

VOCAB = 100000
EMBED = 64
B, L = 4096, 200

NC, NS = 2, 16
NW = NC * NS
BC = B // NW
LH, LL = L // 8, 8
EH, EL = EMBED // 8, 8
BH = B // 128

_mesh = plsc.VectorSubcoreMesh(
    core_axis_name="c", subcore_axis_name="s", num_cores=NC, num_subcores=NS
)


@functools.partial(
    pl.kernel,
    out_type=jax.ShapeDtypeStruct((L, EH, BH, EL, 128), jnp.float32),
    mesh=_mesh,
    compiler_params=pltpu.CompilerParams(
        use_tc_tiling_on_sc=False, needs_layout_passes=False
    ),
    scratch_types=[
        pltpu.VMEM((LH, LL, BC), jnp.int32),
        pltpu.VMEM((L, EMBED), jnp.float32),
        pltpu.VMEM((4, BC, EMBED), jnp.float32),
        pltpu.VMEM((4, EH, EL, 145), jnp.float32),
        pltpu.SemaphoreType.DMA,
        pltpu.SemaphoreType.DMA,
    ],
)
def _sc_embed(x_hbm, tok_hbm, pos_hbm, out_hbm, xcol_v, pos_v, rows_v,
              stg_v, sem_g, sem_o):
    wid = lax.axis_index("s") * NC + lax.axis_index("c")
    pltpu.sync_copy(x_hbm.at[:, wid], xcol_v)
    pltpu.sync_copy(pos_hbm.at[pl.ds(0, L)], pos_v)
    ND = 4

    iv = lax.iota(jnp.int32, 16)
    ivh = iv // EL
    elv = iv % EL

    def start_gather(l, par):
        pltpu.async_copy(
            tok_hbm.at[xcol_v.at[l // 8, l % 8]], rows_v.at[par], sem_g
        )

    def wait_gather(par):
        pltpu.make_async_copy(
            tok_hbm.at[pl.ds(0, BC)], rows_v.at[par], sem_g
        ).wait()

    def wait_out(par):
        pltpu.make_async_copy(
            stg_v.at[par, :, :, pl.ds(0, 128)], out_hbm.at[0, :, wid], sem_o
        ).wait()

    for p in range(ND - 1):
        start_gather(p, p)

    @pl.loop(0, L, step=ND)
    def _block(l0):
        for par in range(ND):
            l = l0 + par
            cur, stg = rows_v.at[par], stg_v.at[par]

            @pl.when(l >= ND)
            def _free_stg():
                wait_out(par)

            @pl.when(l + ND - 1 < L)
            def _prefetch():
                start_gather(l + ND - 1, (par + ND - 1) % ND)

            wait_gather(par)

            pos = [pos_v[l, pl.ds(16 * k, 16)] for k in range(EMBED // 16)]
            ehv = [2 * k + ivh for k in range(EMBED // 16)]

            @plsc.parallel_loop(0, BC, unroll=8)
            def _row(r):
                rv = jnp.broadcast_to(r, (16,)).astype(jnp.int32)
                for k in range(EMBED // 16):
                    val = cur[r, pl.ds(16 * k, 16)] + pos[k]
                    plsc.store_scatter(stg, [ehv[k], elv, rv], val)

            pltpu.async_copy(
                stg.at[:, :, pl.ds(0, 128)], out_hbm.at[l, :, wid], sem_o
            )

    for p in range(ND):
        wait_out(p)


def kernel(x, tok_table, pos_table):
    x4 = x.T.astype(jnp.int32).reshape(LH, LL, BH, 128).transpose(0, 2, 1, 3)
    out5 = _sc_embed(x4, tok_table, pos_table)
    return out5.transpose(2, 4, 0, 1, 3).reshape(B, L, EMBED)
